# baseline (speedup 1.0000x reference)
import functools

import jax
import jax.numpy as jnp
from jax import lax
from jax.experimental import pallas as pl
from jax.experimental.pallas import tpu as pltpu
from jax.experimental.pallas import tpu_sc as plsc

S, B, D, E, K, F = 2048, 1, 1024, 8, 2, 2048
N = S * B
LANES = 128
TM = 128
RMAX = 4992
G = RMAX // TM
NW = 32
CHUNK = N // NW
CS = 256
NCH = N // CS


def _route_kernel(x_ref, wg_ref, pos1_ref, pos2_ref, w1_ref, w2_ref,
                  te_ref):
    xb = x_ref[...]
    logits = lax.dot_general(
        xb.astype(jnp.bfloat16), wg_ref[...].astype(jnp.bfloat16),
        (((1,), (0,)), ((), ())),
        preferred_element_type=jnp.float32)
    lane = lax.broadcasted_iota(jnp.int32, (N, LANES), 1)
    neg = jnp.float32(-1e30)
    logm = jnp.where(lane < E, logits, neg)
    m1 = jnp.max(logm, axis=1, keepdims=True)
    i1 = jnp.min(jnp.where(logm == m1, lane, LANES), axis=1, keepdims=True)
    logm2 = jnp.where(lane == i1, neg, logm)
    m2 = jnp.max(logm2, axis=1, keepdims=True)
    i2 = jnp.min(jnp.where(logm2 == m2, lane, LANES), axis=1, keepdims=True)
    d = jnp.exp(m2 - m1)
    w1_ref[...] = 1.0 / (1.0 + d)
    w2_ref[...] = d / (1.0 + d)

    oh1 = (lane == i1)
    oh2 = (lane == i2)
    oh1b = oh1.astype(jnp.bfloat16)
    oh2b = oh2.astype(jnp.bfloat16)

    r_io = lax.broadcasted_iota(jnp.int32, (CS, CS), 0)
    c_io = lax.broadcasted_iota(jnp.int32, (CS, CS), 1)
    tri = (c_io < r_io).astype(jnp.bfloat16)

    def excl_cumsum(ohb):
        carry = jnp.zeros((1, LANES), jnp.float32)
        parts = []
        for c in range(NCH):
            blk = ohb[c * CS:(c + 1) * CS]
            cc = lax.dot_general(
                tri, blk, (((1,), (0,)), ((), ())),
                preferred_element_type=jnp.float32) + carry
            parts.append(cc)
            carry = carry + jnp.sum(blk.astype(jnp.float32), axis=0,
                                    keepdims=True)
        return jnp.concatenate(parts, axis=0), carry

    cum0, cnt0 = excl_cumsum(oh1b)
    cum1, cnt1 = excl_cumsum(oh2b)

    cnt = (cnt0 + cnt1).astype(jnp.int32)
    pc = ((cnt + (TM - 1)) // TM) * TM
    pcf = pc.astype(jnp.float32)
    r2 = lax.broadcasted_iota(jnp.int32, (LANES, LANES), 0)
    c2 = lax.broadcasted_iota(jnp.int32, (LANES, LANES), 1)
    upper = (r2 < c2).astype(jnp.float32)
    base = lax.dot_general(
        pcf, upper, (((1,), (0,)), ((), ())),
        precision=lax.Precision.HIGHEST,
        preferred_element_type=jnp.float32)

    oh1f = oh1.astype(jnp.float32)
    oh2f = oh2.astype(jnp.float32)
    pos1 = jnp.sum(oh1f * (base + cum0), axis=1, keepdims=True)
    pos2 = jnp.sum(oh2f * (base + cnt0 + cum1), axis=1, keepdims=True)
    pos1_ref[...] = pos1.astype(jnp.int32)
    pos2_ref[...] = pos2.astype(jnp.int32)

    ends = base + pcf
    g_io = lax.broadcasted_iota(jnp.int32, (LANES, LANES), 0)
    l_io = lax.broadcasted_iota(jnp.int32, (LANES, LANES), 1)
    ind = jnp.where(
        (g_io.astype(jnp.float32) * TM >= ends) & (l_io < E), 1, 0)
    te = jnp.minimum(jnp.sum(ind, axis=1, keepdims=True), E - 1)
    te_ref[...] = jnp.broadcast_to(te, (LANES, LANES))


GT = G + N // TM


def _gmm_kernel(te_ref, xs_ref, x_ref, pos1_ref, pos2_ref, wa_ref, wb_ref,
                w1_ref, w2_ref, o_ref):
    del te_ref
    g = pl.program_id(0)
    routed = g < G
    xin = jnp.where(routed, xs_ref[...], x_ref[...])
    xb = xin.astype(jnp.bfloat16)
    h = lax.dot_general(xb, w1_ref[0], (((1,), (0,)), ((), ())),
                        preferred_element_type=jnp.float32)
    h = jnp.maximum(h, 0.0).astype(jnp.bfloat16)
    ye = lax.dot_general(h, w2_ref[0], (((1,), (0,)), ((), ())),
                         preferred_element_type=jnp.float32)
    ye = jnp.where(jnp.abs(ye) < jnp.float32(1e30), ye, 0.0)

    @pl.when(routed)
    def _routed():
        yeb = ye.astype(jnp.bfloat16)
        rel1 = pos1_ref[...] - g * TM
        rel2 = pos2_ref[...] - g * TM
        lane = lax.broadcasted_iota(jnp.int32, (N, TM), 1)
        scat = (jnp.where(lane == rel1, wa_ref[...], 0.0)
                + jnp.where(lane == rel2, wb_ref[...], 0.0))
        contrib = lax.dot_general(
            scat.astype(jnp.bfloat16), yeb, (((1,), (0,)), ((), ())),
            preferred_element_type=jnp.float32)

        @pl.when(g == 0)
        def _init():
            o_ref[...] = contrib

        @pl.when(g != 0)
        def _acc():
            o_ref[...] = o_ref[...] + contrib

    @pl.when(jnp.logical_not(routed))
    def _shared():
        row0 = (g - G) * TM
        o_ref[pl.ds(row0, TM), :] = (o_ref[pl.ds(row0, TM), :]
                                     + (1.0 / K) * ye)


@functools.cache
def _sc_kernels():
    mesh = plsc.VectorSubcoreMesh(core_axis_name="c", subcore_axis_name="s")

    @functools.partial(
        pl.kernel,
        out_type=jax.ShapeDtypeStruct((RMAX, D), jnp.float32),
        mesh=mesh,
        scratch_types=[
            pltpu.VMEM((CHUNK, D), jnp.float32),
            pltpu.VMEM((CHUNK,), jnp.int32),
            pltpu.VMEM((CHUNK,), jnp.int32),
            pltpu.SemaphoreType.DMA,
            pltpu.SemaphoreType.DMA,
        ],
    )
    def _sc_dispatch(x_hbm, pos1_hbm, pos2_hbm, xs_hbm,
                     rows_v, idx1_v, idx2_v, sem1, sem2):
        wid = lax.axis_index("s") * 2 + lax.axis_index("c")
        base = wid * CHUNK
        pltpu.sync_copy(pos1_hbm.at[pl.ds(base, CHUNK)], idx1_v)
        pltpu.sync_copy(pos2_hbm.at[pl.ds(base, CHUNK)], idx2_v)
        pltpu.sync_copy(x_hbm.at[pl.ds(base, CHUNK)], rows_v)
        c1 = pltpu.async_copy(rows_v, xs_hbm.at[idx1_v], sem1)
        c2 = pltpu.async_copy(rows_v, xs_hbm.at[idx2_v], sem2)
        c1.wait()
        c2.wait()

    return _sc_dispatch


def kernel(x, Wg, bg, W1, b1, W2, b2, Ws1, bs1, Ws2, bs2):
    xf = x.reshape(N, D)
    wgp = jnp.zeros((D, LANES), jnp.float32).at[:, :E].set(Wg)

    pos1, pos2, w1c, w2c, te = pl.pallas_call(
        _route_kernel,
        grid=(1,),
        in_specs=[
            pl.BlockSpec((N, D), lambda i: (0, 0)),
            pl.BlockSpec((D, LANES), lambda i: (0, 0)),
        ],
        out_specs=[
            pl.BlockSpec((N, 1), lambda i: (0, 0)),
            pl.BlockSpec((N, 1), lambda i: (0, 0)),
            pl.BlockSpec((N, 1), lambda i: (0, 0)),
            pl.BlockSpec((N, 1), lambda i: (0, 0)),
            pl.BlockSpec((LANES, LANES), lambda i: (0, 0)),
        ],
        out_shape=[
            jax.ShapeDtypeStruct((N, 1), jnp.int32),
            jax.ShapeDtypeStruct((N, 1), jnp.int32),
            jax.ShapeDtypeStruct((N, 1), jnp.float32),
            jax.ShapeDtypeStruct((N, 1), jnp.float32),
            jax.ShapeDtypeStruct((LANES, LANES), jnp.int32),
        ],
    )(xf, wgp)

    pos1r = pos1.reshape(N)
    pos2r = pos2.reshape(N)
    te40 = te[:G, 0]

    sc_dispatch = _sc_kernels()
    xs = sc_dispatch(xf, pos1r, pos2r)

    te_ext = jnp.concatenate(
        [te40, jnp.full((N // TM,), E, jnp.int32)])
    w1a = jnp.concatenate([W1, Ws1[None]], axis=0).astype(jnp.bfloat16)
    w2a = jnp.concatenate([W2, Ws2[None]], axis=0).astype(jnp.bfloat16)

    out = pl.pallas_call(
        _gmm_kernel,
        grid_spec=pltpu.PrefetchScalarGridSpec(
            num_scalar_prefetch=1,
            grid=(GT,),
            in_specs=[
                pl.BlockSpec((TM, D),
                             lambda g, te_s: (jnp.minimum(g, G - 1), 0)),
                pl.BlockSpec((TM, D),
                             lambda g, te_s: (jnp.maximum(g - G, 0), 0)),
                pl.BlockSpec((N, 1), lambda g, te_s: (0, 0)),
                pl.BlockSpec((N, 1), lambda g, te_s: (0, 0)),
                pl.BlockSpec((N, 1), lambda g, te_s: (0, 0)),
                pl.BlockSpec((N, 1), lambda g, te_s: (0, 0)),
                pl.BlockSpec((1, D, F), lambda g, te_s: (te_s[g], 0, 0)),
                pl.BlockSpec((1, F, D), lambda g, te_s: (te_s[g], 0, 0)),
            ],
            out_specs=pl.BlockSpec((N, D), lambda g, te_s: (0, 0)),
        ),
        out_shape=jax.ShapeDtypeStruct((N, D), jnp.float32),
        compiler_params=pltpu.CompilerParams(
            dimension_semantics=("arbitrary",),
        ),
    )(te_ext, xs, xf, pos1, pos2, w1c, w2c, w1a, w2a)
    return out.reshape(S, B, D)


# speedup vs baseline: 1.2318x; 1.2318x over previous
import functools

import jax
import jax.numpy as jnp
from jax import lax
from jax.experimental import pallas as pl
from jax.experimental.pallas import tpu as pltpu
from jax.experimental.pallas import tpu_sc as plsc

S, B, D, E, K, F = 2048, 1, 1024, 8, 2, 2048
N = S * B
LANES = 128
TM = 128
RMAX = 4992
G = RMAX // TM
NW = 32
CHUNK = N // NW
CS = 256
NCH = N // CS


def _route_kernel(x_ref, wg_ref, pos1_ref, pos2_ref, w1_ref, w2_ref,
                  te_ref):
    xb = x_ref[...]
    logits = lax.dot_general(
        xb.astype(jnp.bfloat16), wg_ref[...].astype(jnp.bfloat16),
        (((1,), (0,)), ((), ())),
        preferred_element_type=jnp.float32)
    lane = lax.broadcasted_iota(jnp.int32, (N, LANES), 1)
    neg = jnp.float32(-1e30)
    logm = jnp.where(lane < E, logits, neg)
    m1 = jnp.max(logm, axis=1, keepdims=True)
    i1 = jnp.min(jnp.where(logm == m1, lane, LANES), axis=1, keepdims=True)
    logm2 = jnp.where(lane == i1, neg, logm)
    m2 = jnp.max(logm2, axis=1, keepdims=True)
    i2 = jnp.min(jnp.where(logm2 == m2, lane, LANES), axis=1, keepdims=True)
    d = jnp.exp(m2 - m1)
    w1_ref[...] = 1.0 / (1.0 + d)
    w2_ref[...] = d / (1.0 + d)

    oh1 = (lane == i1)
    oh2 = (lane == i2)
    oh1b = oh1.astype(jnp.bfloat16)
    oh2b = oh2.astype(jnp.bfloat16)

    r_io = lax.broadcasted_iota(jnp.int32, (CS, CS), 0)
    c_io = lax.broadcasted_iota(jnp.int32, (CS, CS), 1)
    tri = (c_io < r_io).astype(jnp.bfloat16)

    def excl_cumsum(ohb):
        carry = jnp.zeros((1, LANES), jnp.float32)
        parts = []
        for c in range(NCH):
            blk = ohb[c * CS:(c + 1) * CS]
            cc = lax.dot_general(
                tri, blk, (((1,), (0,)), ((), ())),
                preferred_element_type=jnp.float32) + carry
            parts.append(cc)
            carry = carry + jnp.sum(blk.astype(jnp.float32), axis=0,
                                    keepdims=True)
        return jnp.concatenate(parts, axis=0), carry

    cum0, cnt0 = excl_cumsum(oh1b)
    cum1, cnt1 = excl_cumsum(oh2b)

    cnt = (cnt0 + cnt1).astype(jnp.int32)
    pc = ((cnt + (TM - 1)) // TM) * TM
    pcf = pc.astype(jnp.float32)
    r2 = lax.broadcasted_iota(jnp.int32, (LANES, LANES), 0)
    c2 = lax.broadcasted_iota(jnp.int32, (LANES, LANES), 1)
    upper = (r2 < c2).astype(jnp.float32)
    base = lax.dot_general(
        pcf, upper, (((1,), (0,)), ((), ())),
        precision=lax.Precision.HIGHEST,
        preferred_element_type=jnp.float32)

    oh1f = oh1.astype(jnp.float32)
    oh2f = oh2.astype(jnp.float32)
    pos1 = jnp.sum(oh1f * (base + cum0), axis=1, keepdims=True)
    pos2 = jnp.sum(oh2f * (base + cnt0 + cum1), axis=1, keepdims=True)
    pos1_ref[...] = pos1.astype(jnp.int32)
    pos2_ref[...] = pos2.astype(jnp.int32)

    ends = base + pcf
    g_io = lax.broadcasted_iota(jnp.int32, (LANES, LANES), 0)
    l_io = lax.broadcasted_iota(jnp.int32, (LANES, LANES), 1)
    ind = jnp.where(
        (g_io.astype(jnp.float32) * TM >= ends) & (l_io < E), 1, 0)
    te = jnp.minimum(jnp.sum(ind, axis=1, keepdims=True), E - 1)
    te_ref[...] = jnp.broadcast_to(te, (LANES, LANES))


def _final_kernel(x_ref, w1_ref, w2_ref, r_ref, o_ref):
    xb = x_ref[...].astype(jnp.bfloat16)
    h = lax.dot_general(xb, w1_ref[...], (((1,), (0,)), ((), ())),
                        preferred_element_type=jnp.float32)
    h = jnp.maximum(h, 0.0).astype(jnp.bfloat16)
    y = lax.dot_general(h, w2_ref[...], (((1,), (0,)), ((), ())),
                        preferred_element_type=jnp.float32)
    o_ref[...] = (1.0 / K) * y + r_ref[...]


def _gmm_kernel(te_ref, xs_ref, pos1_ref, pos2_ref, wa_ref, wb_ref,
                w1_ref, w2_ref, o_ref):
    del te_ref
    g = pl.program_id(0)
    xb = xs_ref[...].astype(jnp.bfloat16)
    h = lax.dot_general(xb, w1_ref[0], (((1,), (0,)), ((), ())),
                        preferred_element_type=jnp.float32)
    h = jnp.maximum(h, 0.0).astype(jnp.bfloat16)
    ye = lax.dot_general(h, w2_ref[0], (((1,), (0,)), ((), ())),
                         preferred_element_type=jnp.float32)
    ye = jnp.where(jnp.abs(ye) < jnp.float32(1e30), ye, 0.0)
    yeb = ye.astype(jnp.bfloat16)
    rel1 = pos1_ref[...] - g * TM
    rel2 = pos2_ref[...] - g * TM
    lane = lax.broadcasted_iota(jnp.int32, (N, TM), 1)
    scat = (jnp.where(lane == rel1, wa_ref[...], 0.0)
            + jnp.where(lane == rel2, wb_ref[...], 0.0))
    contrib = lax.dot_general(
        scat.astype(jnp.bfloat16), yeb, (((1,), (0,)), ((), ())),
        preferred_element_type=jnp.float32)

    @pl.when(g == 0)
    def _init():
        o_ref[...] = contrib

    @pl.when(g != 0)
    def _acc():
        o_ref[...] = o_ref[...] + contrib


@functools.cache
def _sc_kernels():
    mesh = plsc.VectorSubcoreMesh(core_axis_name="c", subcore_axis_name="s")

    @functools.partial(
        pl.kernel,
        out_type=jax.ShapeDtypeStruct((RMAX, D), jnp.float32),
        mesh=mesh,
        scratch_types=[
            pltpu.VMEM((CHUNK, D), jnp.float32),
            pltpu.VMEM((CHUNK,), jnp.int32),
            pltpu.VMEM((CHUNK,), jnp.int32),
            pltpu.SemaphoreType.DMA,
            pltpu.SemaphoreType.DMA,
        ],
    )
    def _sc_dispatch(x_hbm, pos1_hbm, pos2_hbm, xs_hbm,
                     rows_v, idx1_v, idx2_v, sem1, sem2):
        wid = lax.axis_index("s") * 2 + lax.axis_index("c")
        base = wid * CHUNK
        pltpu.sync_copy(pos1_hbm.at[pl.ds(base, CHUNK)], idx1_v)
        pltpu.sync_copy(pos2_hbm.at[pl.ds(base, CHUNK)], idx2_v)
        pltpu.sync_copy(x_hbm.at[pl.ds(base, CHUNK)], rows_v)
        c1 = pltpu.async_copy(rows_v, xs_hbm.at[idx1_v], sem1)
        c2 = pltpu.async_copy(rows_v, xs_hbm.at[idx2_v], sem2)
        c1.wait()
        c2.wait()

    return _sc_dispatch


def kernel(x, Wg, bg, W1, b1, W2, b2, Ws1, bs1, Ws2, bs2):
    xf = x.reshape(N, D)
    wgp = jnp.zeros((D, LANES), jnp.float32).at[:, :E].set(Wg)

    pos1, pos2, w1c, w2c, te = pl.pallas_call(
        _route_kernel,
        grid=(1,),
        in_specs=[
            pl.BlockSpec((N, D), lambda i: (0, 0)),
            pl.BlockSpec((D, LANES), lambda i: (0, 0)),
        ],
        out_specs=[
            pl.BlockSpec((N, 1), lambda i: (0, 0)),
            pl.BlockSpec((N, 1), lambda i: (0, 0)),
            pl.BlockSpec((N, 1), lambda i: (0, 0)),
            pl.BlockSpec((N, 1), lambda i: (0, 0)),
            pl.BlockSpec((LANES, LANES), lambda i: (0, 0)),
        ],
        out_shape=[
            jax.ShapeDtypeStruct((N, 1), jnp.int32),
            jax.ShapeDtypeStruct((N, 1), jnp.int32),
            jax.ShapeDtypeStruct((N, 1), jnp.float32),
            jax.ShapeDtypeStruct((N, 1), jnp.float32),
            jax.ShapeDtypeStruct((LANES, LANES), jnp.int32),
        ],
    )(xf, wgp)

    pos1r = pos1.reshape(N)
    pos2r = pos2.reshape(N)
    te40 = te[:G, 0]

    sc_dispatch = _sc_kernels()
    xs = sc_dispatch(xf, pos1r, pos2r)

    outr = pl.pallas_call(
        _gmm_kernel,
        grid_spec=pltpu.PrefetchScalarGridSpec(
            num_scalar_prefetch=1,
            grid=(G,),
            in_specs=[
                pl.BlockSpec((TM, D), lambda g, te_s: (g, 0)),
                pl.BlockSpec((N, 1), lambda g, te_s: (0, 0)),
                pl.BlockSpec((N, 1), lambda g, te_s: (0, 0)),
                pl.BlockSpec((N, 1), lambda g, te_s: (0, 0)),
                pl.BlockSpec((N, 1), lambda g, te_s: (0, 0)),
                pl.BlockSpec((1, D, F), lambda g, te_s: (te_s[g], 0, 0)),
                pl.BlockSpec((1, F, D), lambda g, te_s: (te_s[g], 0, 0)),
            ],
            out_specs=pl.BlockSpec((N, D), lambda g, te_s: (0, 0)),
        ),
        out_shape=jax.ShapeDtypeStruct((N, D), jnp.float32),
        compiler_params=pltpu.CompilerParams(
            dimension_semantics=("arbitrary",),
        ),
    )(te40, xs, pos1, pos2, w1c, w2c,
      W1.astype(jnp.bfloat16), W2.astype(jnp.bfloat16))

    out = pl.pallas_call(
        _final_kernel,
        grid=(2,),
        in_specs=[
            pl.BlockSpec((N // 2, D), lambda i: (i, 0)),
            pl.BlockSpec((D, F), lambda i: (0, 0)),
            pl.BlockSpec((F, D), lambda i: (0, 0)),
            pl.BlockSpec((N // 2, D), lambda i: (i, 0)),
        ],
        out_specs=pl.BlockSpec((N // 2, D), lambda i: (i, 0)),
        out_shape=jax.ShapeDtypeStruct((N, D), jnp.float32),
    )(xf, Ws1.astype(jnp.bfloat16), Ws2.astype(jnp.bfloat16), outr)
    return out.reshape(S, B, D)
